# baseline (speedup 1.0000x reference)
import functools

import jax
import jax.numpy as jnp
from jax import lax
from jax.experimental import pallas as pl
from jax.experimental.pallas import tpu as pltpu
from jax.experimental.pallas import tpu_sc as plsc

N_NODES = 10000
N_EDGES = 320000
NUM_GRAPHS = 256
HID = 128
NC, NS = 2, 16
NW = NC * NS
EPT = N_EDGES // NW
K = 40
NCHUNK = EPT // K
N_P = 10240
RPT = N_P // NS

_SC_MESH = plsc.VectorSubcoreMesh(core_axis_name="c", subcore_axis_name="s")


NBUF = 5
NIDX = 10
LOOK_G = 2
LOOK_I = 4
NTRIP = NCHUNK // NIDX


def _deg_body(ei3_hbm, zeros_hbm, ones_hbm, out_hbm, idxv, ones_v, acc, sem):
    c = lax.axis_index("c")
    s = lax.axis_index("s")
    wid = c * NS + s
    pltpu.sync_copy(ones_hbm, ones_v)
    pltpu.sync_copy(ei3_hbm.at[wid], idxv)
    pltpu.sync_copy(zeros_hbm.at[pl.ds(s * RPT, RPT)], acc.at[pl.ds(s * RPT, RPT)])
    plsc.subcore_barrier()

    def fire(i, carry):
        pltpu.async_copy(ones_v, acc.at[idxv.at[i, 1]], sem, add=True)
        return carry

    lax.fori_loop(0, NCHUNK, fire, 0)

    def drain(i, carry):
        pltpu.make_async_copy(ones_v, acc.at[idxv.at[0, 1]], sem).wait()
        return carry

    lax.fori_loop(0, NCHUNK, drain, 0)
    plsc.subcore_barrier()
    pltpu.sync_copy(acc.at[pl.ds(s * RPT, RPT)], out_hbm.at[c].at[pl.ds(s * RPT, RPT)])


_deg_kernel = pl.kernel(
    _deg_body,
    out_type=jax.ShapeDtypeStruct((NC, N_P), jnp.float32),
    mesh=_SC_MESH,
    scratch_types=[
        pltpu.VMEM((NCHUNK, 2, K), jnp.int32),
        pltpu.VMEM((K,), jnp.float32),
        pltpu.VMEM_SHARED((N_P,), jnp.float32),
        pltpu.SemaphoreType.DMA,
    ],
)


def _edge_body(g_hbm, ei3_hbm, out_hbm, idxv, rows, acc, *sems):
    gsem, ssem, isem = sems[:NBUF], sems[NBUF:2 * NBUF], sems[2 * NBUF:]
    c = lax.axis_index("c")
    s = lax.axis_index("s")
    wid = c * NS + s
    pltpu.sync_copy(g_hbm.at[pl.ds(s * RPT, RPT)], acc.at[pl.ds(s * RPT, RPT)])
    plsc.subcore_barrier()

    def idx_load(i, q):
        pltpu.async_copy(ei3_hbm.at[wid].at[i], idxv.at[q], isem[q])

    def idx_wait(q):
        pltpu.make_async_copy(ei3_hbm.at[wid].at[0], idxv.at[q], isem[q]).wait()

    def gather(i, q, b):
        pltpu.async_copy(g_hbm.at[idxv.at[q, 0]], rows.at[b], gsem[b])

    def gather_wait(b):
        pltpu.make_async_copy(g_hbm.at[idxv.at[0, 0]], rows.at[b], gsem[b]).wait()

    def scatter(i, q, b):
        pltpu.async_copy(rows.at[b], acc.at[idxv.at[q, 1]], ssem[b], add=True)

    def scatter_wait(b):
        pltpu.make_async_copy(rows.at[b], acc.at[idxv.at[0, 1]], ssem[b]).wait()

    for i in range(LOOK_I):
        idx_load(i, i)
    for i in range(LOOK_G):
        idx_wait(i)
        gather(i, i, i)

    def trip(t, carry):
        for u in range(NIDX):
            i = t * NIDX + u
            @pl.when(i + LOOK_I < NCHUNK)
            def _():
                idx_load(i + LOOK_I, (u + LOOK_I) % NIDX)

            bg = (u + LOOK_G) % NBUF
            @pl.when(jnp.logical_and(i + LOOK_G < NCHUNK, i + LOOK_G >= NBUF))
            def _():
                scatter_wait(bg)

            @pl.when(i + LOOK_G < NCHUNK)
            def _():
                idx_wait((u + LOOK_G) % NIDX)
                gather(i + LOOK_G, (u + LOOK_G) % NIDX, bg)

            gather_wait(u % NBUF)
            scatter(i, u % NIDX, u % NBUF)
        return carry

    lax.fori_loop(0, NTRIP, trip, 0)
    for b in range(NBUF):
        scatter_wait(b)
    plsc.subcore_barrier()
    pltpu.sync_copy(acc.at[pl.ds(s * RPT, RPT)], out_hbm.at[c].at[pl.ds(s * RPT, RPT)])


_edge_kernel = pl.kernel(
    _edge_body,
    out_type=jax.ShapeDtypeStruct((NC, N_P, HID), jnp.float32),
    mesh=_SC_MESH,
    scratch_types=[
        pltpu.VMEM((NIDX, 2, K), jnp.int32),
        pltpu.VMEM((NBUF, K, HID), jnp.float32),
        pltpu.VMEM_SHARED((N_P, HID), jnp.float32),
    ] + [pltpu.SemaphoreType.DMA] * (2 * NBUF + NIDX),
)


def _row_mask():
    return (lax.broadcasted_iota(jnp.int32, (N_P, 1), 0) < N_NODES
            ).astype(jnp.float32)


def _gcn_bn_relu(p_ref, g_ref, dinv_ref, b_ref, bng_ref, bnb_ref):
    mask = _row_mask()
    t = ((p_ref[0] + p_ref[1] - g_ref[...]) * dinv_ref[...] + b_ref[...]) * mask
    inv_n = 1.0 / N_NODES
    mu = jnp.sum(t, axis=0, keepdims=True) * inv_n
    d = (t - mu) * mask
    var = jnp.sum(d * d, axis=0, keepdims=True) * inv_n
    return jnp.maximum((t - mu) * lax.rsqrt(var + 1e-5) * bng_ref[...] + bnb_ref[...], 0.0)


def _tc0_body(x_ref, w_ref, dinv_ref, g_ref):
    g_ref[...] = jnp.dot(x_ref[...], w_ref[...],
                         preferred_element_type=jnp.float32) * dinv_ref[...]


def _tc_mid_body(p_ref, g_ref, dinv_ref, b_ref, bng_ref, bnb_ref, w_ref, out_ref):
    u = _gcn_bn_relu(p_ref, g_ref, dinv_ref, b_ref, bng_ref, bnb_ref)
    out_ref[...] = jnp.dot(u, w_ref[...],
                           preferred_element_type=jnp.float32) * dinv_ref[...]


def _tc_fin_body(p_ref, g_ref, dinv_ref, b_ref, bng_ref, bnb_ref, batch_ref, pf_ref,
                 pw1, pb1, pw2, pb2,
                 cw1a, cw1b, cb1, cw2, cb2,
                 aw1a0, aw1b0, ab10, aw20, ab20,
                 aw1a1, aw1b1, ab11, aw21, ab21,
                 logits_ref, a0_ref, a1_ref):
    u = _gcn_bn_relu(p_ref, g_ref, dinv_ref, b_ref, bng_ref, bnb_ref)

    onehot = (batch_ref[...] == lax.broadcasted_iota(jnp.int32, (1, NUM_GRAPHS), 1)
              ).astype(jnp.float32)
    dn = (((0,), (0,)), ((), ()))
    hp = jax.lax.Precision.HIGHEST
    sums = lax.dot_general(onehot, u, dn, precision=hp,
                           preferred_element_type=jnp.float32)
    counts = lax.dot_general(onehot, jnp.ones((N_P, 1), jnp.float32), dn, precision=hp,
                             preferred_element_type=jnp.float32)
    ligand = sums / jnp.maximum(counts, 1.0)

    pk = jnp.maximum(jnp.dot(pf_ref[...], pw1[...],
                             preferred_element_type=jnp.float32) + pb1[...], 0.0)
    pk2 = jnp.dot(pk, pw2[...], preferred_element_type=jnp.float32) + pb2[...]

    def head(w1a, w1b, b1, w2, b2):
        h = jnp.maximum(
            jnp.dot(ligand, w1a[...], preferred_element_type=jnp.float32)
            + jnp.dot(pk2, w1b[...], preferred_element_type=jnp.float32)
            + b1[...], 0.0)
        return jnp.dot(h, w2[...], preferred_element_type=jnp.float32) + b2[...]

    logits_ref[...] = head(cw1a, cw1b, cb1, cw2, cb2)
    a0_ref[...] = head(aw1a0, aw1b0, ab10, aw20, ab20)
    a1_ref[...] = head(aw1a1, aw1b1, ab11, aw21, ab21)


_tc0 = pl.pallas_call(
    _tc0_body, out_shape=jax.ShapeDtypeStruct((N_P, HID), jnp.float32))

_tc_mid = pl.pallas_call(
    _tc_mid_body, out_shape=jax.ShapeDtypeStruct((N_P, HID), jnp.float32))

_tc_fin = pl.pallas_call(
    _tc_fin_body,
    out_shape=[jax.ShapeDtypeStruct((NUM_GRAPHS, 1), jnp.float32)] * 3)


def kernel(x, edge_index, batch, pocket_features, params):
    src = edge_index[0]
    dst = edge_index[1]

    ei3 = jnp.stack([src.reshape(NW, NCHUNK, K), dst.reshape(NW, NCHUNK, K)],
                    axis=2)
    deg_parts = _deg_kernel(ei3, jnp.zeros((N_P,), jnp.float32),
                            jnp.ones((K,), jnp.float32))
    deg = deg_parts[0] + deg_parts[1] + 1.0
    dinv = lax.rsqrt(deg)[:, None]

    x_pad = jnp.pad(x, ((0, N_P - N_NODES), (0, 1)))
    w1_pad = jnp.pad(params["conv_w"][0], ((0, 1), (0, 0)))
    g = _tc0(x_pad, w1_pad, dinv)

    row = lambda v: v[None, :]
    for i in range(2):
        parts = _edge_kernel(g, ei3)
        g = _tc_mid(parts, g, dinv, row(params["conv_b"][i]),
                    row(params["bn_g"][i]), row(params["bn_b"][i]),
                    params["conv_w"][i + 1])

    parts = _edge_kernel(g, ei3)
    batch_pad = jnp.pad(batch, (0, N_P - N_NODES), constant_values=-1)
    logits, a0, a1 = _tc_fin(
        parts, g, dinv, row(params["conv_b"][2]),
        row(params["bn_g"][2]), row(params["bn_b"][2]),
        batch_pad[:, None], pocket_features[None, :],
        params["pocket_w1"], row(params["pocket_b1"]),
        params["pocket_w2"], row(params["pocket_b2"]),
        params["cls_w1"][:HID], params["cls_w1"][HID:], row(params["cls_b1"]),
        params["cls_w2"], row(params["cls_b2"]),
        params["aux_w1"][0][:HID], params["aux_w1"][0][HID:], row(params["aux_b1"][0]),
        params["aux_w2"][0], row(params["aux_b2"][0]),
        params["aux_w1"][1][:HID], params["aux_w1"][1][HID:], row(params["aux_b1"][1]),
        params["aux_w2"][1], row(params["aux_b2"][1]),
    )
    return logits[:, 0], a0[:, 0], a1[:, 0], jnp.float32(0.0)


# speedup vs baseline: 1.0658x; 1.0658x over previous
import functools

import jax
import jax.numpy as jnp
from jax import lax
from jax.experimental import pallas as pl
from jax.experimental.pallas import tpu as pltpu
from jax.experimental.pallas import tpu_sc as plsc

N_NODES = 10000
N_EDGES = 320000
NUM_GRAPHS = 256
HID = 128
NC, NS = 2, 16
NW = NC * NS
EPT = N_EDGES // NW
K = 40
NCHUNK = EPT // K
N_P = 10240
RPT = N_P // NS

_SC_MESH = plsc.VectorSubcoreMesh(core_axis_name="c", subcore_axis_name="s")


NBUF = 5
NIDX = 10
LOOK_G = 3
LOOK_I = 6
NTRIP = NCHUNK // NIDX


def _deg_body(ei3_hbm, zeros_hbm, ones_hbm, out_hbm, idxv, ones_v, acc, sem):
    c = lax.axis_index("c")
    s = lax.axis_index("s")
    wid = c * NS + s
    pltpu.sync_copy(ones_hbm, ones_v)
    pltpu.sync_copy(ei3_hbm.at[wid], idxv)
    pltpu.sync_copy(zeros_hbm.at[pl.ds(s * RPT, RPT)], acc.at[pl.ds(s * RPT, RPT)])
    plsc.subcore_barrier()

    def fire(i, carry):
        pltpu.async_copy(ones_v, acc.at[idxv.at[i, 1]], sem, add=True)
        return carry

    lax.fori_loop(0, NCHUNK, fire, 0)

    def drain(i, carry):
        pltpu.make_async_copy(ones_v, acc.at[idxv.at[0, 1]], sem).wait()
        return carry

    lax.fori_loop(0, NCHUNK, drain, 0)
    plsc.subcore_barrier()
    pltpu.sync_copy(acc.at[pl.ds(s * RPT, RPT)], out_hbm.at[c].at[pl.ds(s * RPT, RPT)])


_deg_kernel = pl.kernel(
    _deg_body,
    out_type=jax.ShapeDtypeStruct((NC, N_P), jnp.float32),
    mesh=_SC_MESH,
    scratch_types=[
        pltpu.VMEM((NCHUNK, 2, K), jnp.int32),
        pltpu.VMEM((K,), jnp.float32),
        pltpu.VMEM_SHARED((N_P,), jnp.float32),
        pltpu.SemaphoreType.DMA,
    ],
)


def _edge_body(g_hbm, ei3_hbm, out_hbm, idxv, rows, acc, *sems):
    gsem, ssem, isem = sems[:NBUF], sems[NBUF:2 * NBUF], sems[2 * NBUF:]
    c = lax.axis_index("c")
    s = lax.axis_index("s")
    wid = c * NS + s
    pltpu.sync_copy(g_hbm.at[pl.ds(s * RPT, RPT)], acc.at[pl.ds(s * RPT, RPT)])
    plsc.subcore_barrier()

    def idx_load(i, q):
        pltpu.async_copy(ei3_hbm.at[wid].at[i], idxv.at[q], isem[q])

    def idx_wait(q):
        pltpu.make_async_copy(ei3_hbm.at[wid].at[0], idxv.at[q], isem[q]).wait()

    def gather(i, q, b):
        pltpu.async_copy(g_hbm.at[idxv.at[q, 0]], rows.at[b], gsem[b])

    def gather_wait(b):
        pltpu.make_async_copy(g_hbm.at[idxv.at[0, 0]], rows.at[b], gsem[b]).wait()

    def scatter(i, q, b):
        pltpu.async_copy(rows.at[b], acc.at[idxv.at[q, 1]], ssem[b], add=True)

    def scatter_wait(b):
        pltpu.make_async_copy(rows.at[b], acc.at[idxv.at[0, 1]], ssem[b]).wait()

    for i in range(LOOK_I):
        idx_load(i, i)
    for i in range(LOOK_G):
        idx_wait(i)
        gather(i, i, i)

    def trip(t, carry):
        for u in range(NIDX):
            i = t * NIDX + u
            @pl.when(i + LOOK_I < NCHUNK)
            def _():
                idx_load(i + LOOK_I, (u + LOOK_I) % NIDX)

            bg = (u + LOOK_G) % NBUF
            @pl.when(jnp.logical_and(i + LOOK_G < NCHUNK, i + LOOK_G >= NBUF))
            def _():
                scatter_wait(bg)

            @pl.when(i + LOOK_G < NCHUNK)
            def _():
                idx_wait((u + LOOK_G) % NIDX)
                gather(i + LOOK_G, (u + LOOK_G) % NIDX, bg)

            gather_wait(u % NBUF)
            scatter(i, u % NIDX, u % NBUF)
        return carry

    lax.fori_loop(0, NTRIP, trip, 0)
    for b in range(NBUF):
        scatter_wait(b)
    plsc.subcore_barrier()
    pltpu.sync_copy(acc.at[pl.ds(s * RPT, RPT)], out_hbm.at[c].at[pl.ds(s * RPT, RPT)])


_edge_kernel = pl.kernel(
    _edge_body,
    out_type=jax.ShapeDtypeStruct((NC, N_P, HID), jnp.float32),
    mesh=_SC_MESH,
    scratch_types=[
        pltpu.VMEM((NIDX, 2, K), jnp.int32),
        pltpu.VMEM((NBUF, K, HID), jnp.float32),
        pltpu.VMEM_SHARED((N_P, HID), jnp.float32),
    ] + [pltpu.SemaphoreType.DMA] * (2 * NBUF + NIDX),
)


def _row_mask():
    return (lax.broadcasted_iota(jnp.int32, (N_P, 1), 0) < N_NODES
            ).astype(jnp.float32)


def _gcn_bn_relu(p_ref, g_ref, dinv_ref, b_ref, bng_ref, bnb_ref):
    mask = _row_mask()
    t = ((p_ref[0] + p_ref[1] - g_ref[...]) * dinv_ref[...] + b_ref[...]) * mask
    inv_n = 1.0 / N_NODES
    mu = jnp.sum(t, axis=0, keepdims=True) * inv_n
    d = (t - mu) * mask
    var = jnp.sum(d * d, axis=0, keepdims=True) * inv_n
    return jnp.maximum((t - mu) * lax.rsqrt(var + 1e-5) * bng_ref[...] + bnb_ref[...], 0.0)


def _tc0_body(x_ref, w_ref, dinv_ref, g_ref):
    g_ref[...] = jnp.dot(x_ref[...], w_ref[...],
                         preferred_element_type=jnp.float32) * dinv_ref[...]


def _tc_mid_body(p_ref, g_ref, dinv_ref, b_ref, bng_ref, bnb_ref, w_ref, out_ref):
    u = _gcn_bn_relu(p_ref, g_ref, dinv_ref, b_ref, bng_ref, bnb_ref)
    out_ref[...] = jnp.dot(u, w_ref[...],
                           preferred_element_type=jnp.float32) * dinv_ref[...]


def _tc_fin_body(p_ref, g_ref, dinv_ref, b_ref, bng_ref, bnb_ref, batch_ref, pf_ref,
                 pw1, pb1, pw2, pb2,
                 cw1a, cw1b, cb1, cw2, cb2,
                 aw1a0, aw1b0, ab10, aw20, ab20,
                 aw1a1, aw1b1, ab11, aw21, ab21,
                 logits_ref, a0_ref, a1_ref):
    u = _gcn_bn_relu(p_ref, g_ref, dinv_ref, b_ref, bng_ref, bnb_ref)

    onehot = (batch_ref[...] == lax.broadcasted_iota(jnp.int32, (1, NUM_GRAPHS), 1)
              ).astype(jnp.float32)
    dn = (((0,), (0,)), ((), ()))
    hp = jax.lax.Precision.HIGHEST
    sums = lax.dot_general(onehot, u, dn, precision=hp,
                           preferred_element_type=jnp.float32)
    counts = lax.dot_general(onehot, jnp.ones((N_P, 1), jnp.float32), dn, precision=hp,
                             preferred_element_type=jnp.float32)
    ligand = sums / jnp.maximum(counts, 1.0)

    pk = jnp.maximum(jnp.dot(pf_ref[...], pw1[...],
                             preferred_element_type=jnp.float32) + pb1[...], 0.0)
    pk2 = jnp.dot(pk, pw2[...], preferred_element_type=jnp.float32) + pb2[...]

    def head(w1a, w1b, b1, w2, b2):
        h = jnp.maximum(
            jnp.dot(ligand, w1a[...], preferred_element_type=jnp.float32)
            + jnp.dot(pk2, w1b[...], preferred_element_type=jnp.float32)
            + b1[...], 0.0)
        return jnp.dot(h, w2[...], preferred_element_type=jnp.float32) + b2[...]

    logits_ref[...] = head(cw1a, cw1b, cb1, cw2, cb2)
    a0_ref[...] = head(aw1a0, aw1b0, ab10, aw20, ab20)
    a1_ref[...] = head(aw1a1, aw1b1, ab11, aw21, ab21)


_tc0 = pl.pallas_call(
    _tc0_body, out_shape=jax.ShapeDtypeStruct((N_P, HID), jnp.float32))

_tc_mid = pl.pallas_call(
    _tc_mid_body, out_shape=jax.ShapeDtypeStruct((N_P, HID), jnp.float32))

_tc_fin = pl.pallas_call(
    _tc_fin_body,
    out_shape=[jax.ShapeDtypeStruct((NUM_GRAPHS, 1), jnp.float32)] * 3)


def kernel(x, edge_index, batch, pocket_features, params):
    src = edge_index[0]
    dst = edge_index[1]

    ei3 = jnp.stack([src.reshape(NW, NCHUNK, K), dst.reshape(NW, NCHUNK, K)],
                    axis=2)
    deg_parts = _deg_kernel(ei3, jnp.zeros((N_P,), jnp.float32),
                            jnp.ones((K,), jnp.float32))
    deg = deg_parts[0] + deg_parts[1] + 1.0
    dinv = lax.rsqrt(deg)[:, None]

    x_pad = jnp.pad(x, ((0, N_P - N_NODES), (0, 1)))
    w1_pad = jnp.pad(params["conv_w"][0], ((0, 1), (0, 0)))
    g = _tc0(x_pad, w1_pad, dinv)

    row = lambda v: v[None, :]
    for i in range(2):
        parts = _edge_kernel(g, ei3)
        g = _tc_mid(parts, g, dinv, row(params["conv_b"][i]),
                    row(params["bn_g"][i]), row(params["bn_b"][i]),
                    params["conv_w"][i + 1])

    parts = _edge_kernel(g, ei3)
    batch_pad = jnp.pad(batch, (0, N_P - N_NODES), constant_values=-1)
    logits, a0, a1 = _tc_fin(
        parts, g, dinv, row(params["conv_b"][2]),
        row(params["bn_g"][2]), row(params["bn_b"][2]),
        batch_pad[:, None], pocket_features[None, :],
        params["pocket_w1"], row(params["pocket_b1"]),
        params["pocket_w2"], row(params["pocket_b2"]),
        params["cls_w1"][:HID], params["cls_w1"][HID:], row(params["cls_b1"]),
        params["cls_w2"], row(params["cls_b2"]),
        params["aux_w1"][0][:HID], params["aux_w1"][0][HID:], row(params["aux_b1"][0]),
        params["aux_w2"][0], row(params["aux_b2"][0]),
        params["aux_w1"][1][:HID], params["aux_w1"][1][HID:], row(params["aux_b1"][1]),
        params["aux_w2"][1], row(params["aux_b2"][1]),
    )
    return logits[:, 0], a0[:, 0], a1[:, 0], jnp.float32(0.0)


# speedup vs baseline: 1.0998x; 1.0319x over previous
import functools

import jax
import jax.numpy as jnp
from jax import lax
from jax.experimental import pallas as pl
from jax.experimental.pallas import tpu as pltpu
from jax.experimental.pallas import tpu_sc as plsc

N_NODES = 10000
N_EDGES = 320000
NUM_GRAPHS = 256
HID = 128
NC, NS = 2, 16
NW = NC * NS
EPT = N_EDGES // NW
K = 40
NCHUNK = EPT // K
N_P = 10240
RPT = N_P // NS

_SC_MESH = plsc.VectorSubcoreMesh(core_axis_name="c", subcore_axis_name="s")


NBUF = 5
NIDX = 10
LOOK_G = 4
LOOK_I = 8
NTRIP = NCHUNK // NIDX


def _deg_body(ei3_hbm, zeros_hbm, ones_hbm, out_hbm, idxv, ones_v, acc, sem):
    c = lax.axis_index("c")
    s = lax.axis_index("s")
    wid = c * NS + s
    pltpu.sync_copy(ones_hbm, ones_v)
    pltpu.sync_copy(ei3_hbm.at[wid], idxv)
    pltpu.sync_copy(zeros_hbm.at[pl.ds(s * RPT, RPT)], acc.at[pl.ds(s * RPT, RPT)])
    plsc.subcore_barrier()

    def fire(i, carry):
        pltpu.async_copy(ones_v, acc.at[idxv.at[i, 1]], sem, add=True)
        return carry

    lax.fori_loop(0, NCHUNK, fire, 0)

    def drain(i, carry):
        pltpu.make_async_copy(ones_v, acc.at[idxv.at[0, 1]], sem).wait()
        return carry

    lax.fori_loop(0, NCHUNK, drain, 0)
    plsc.subcore_barrier()
    pltpu.sync_copy(acc.at[pl.ds(s * RPT, RPT)], out_hbm.at[c].at[pl.ds(s * RPT, RPT)])


_deg_kernel = pl.kernel(
    _deg_body,
    out_type=jax.ShapeDtypeStruct((NC, N_P), jnp.float32),
    mesh=_SC_MESH,
    scratch_types=[
        pltpu.VMEM((NCHUNK, 2, K), jnp.int32),
        pltpu.VMEM((K,), jnp.float32),
        pltpu.VMEM_SHARED((N_P,), jnp.float32),
        pltpu.SemaphoreType.DMA,
    ],
)


def _edge_body(g_hbm, ei3_hbm, out_hbm, idxv, rows, acc, *sems):
    gsem, ssem, isem = sems[:NBUF], sems[NBUF:2 * NBUF], sems[2 * NBUF:]
    c = lax.axis_index("c")
    s = lax.axis_index("s")
    wid = c * NS + s
    pltpu.sync_copy(g_hbm.at[pl.ds(s * RPT, RPT)], acc.at[pl.ds(s * RPT, RPT)])
    plsc.subcore_barrier()

    def idx_load(i, q):
        pltpu.async_copy(ei3_hbm.at[wid].at[i], idxv.at[q], isem[q])

    def idx_wait(q):
        pltpu.make_async_copy(ei3_hbm.at[wid].at[0], idxv.at[q], isem[q]).wait()

    def gather(i, q, b):
        pltpu.async_copy(g_hbm.at[idxv.at[q, 0]], rows.at[b], gsem[b])

    def gather_wait(b):
        pltpu.make_async_copy(g_hbm.at[idxv.at[0, 0]], rows.at[b], gsem[b]).wait()

    def scatter(i, q, b):
        pltpu.async_copy(rows.at[b], acc.at[idxv.at[q, 1]], ssem[b], add=True)

    def scatter_wait(b):
        pltpu.make_async_copy(rows.at[b], acc.at[idxv.at[0, 1]], ssem[b]).wait()

    for i in range(LOOK_I):
        idx_load(i, i)
    for i in range(LOOK_G):
        idx_wait(i)
        gather(i, i, i)

    def trip(t, carry):
        for u in range(NIDX):
            i = t * NIDX + u
            @pl.when(i + LOOK_I < NCHUNK)
            def _():
                idx_load(i + LOOK_I, (u + LOOK_I) % NIDX)

            bg = (u + LOOK_G) % NBUF
            @pl.when(jnp.logical_and(i + LOOK_G < NCHUNK, i + LOOK_G >= NBUF))
            def _():
                scatter_wait(bg)

            @pl.when(i + LOOK_G < NCHUNK)
            def _():
                idx_wait((u + LOOK_G) % NIDX)
                gather(i + LOOK_G, (u + LOOK_G) % NIDX, bg)

            gather_wait(u % NBUF)
            scatter(i, u % NIDX, u % NBUF)
        return carry

    lax.fori_loop(0, NTRIP, trip, 0)
    for b in range(NBUF):
        scatter_wait(b)
    plsc.subcore_barrier()
    pltpu.sync_copy(acc.at[pl.ds(s * RPT, RPT)], out_hbm.at[c].at[pl.ds(s * RPT, RPT)])


_edge_kernel = pl.kernel(
    _edge_body,
    out_type=jax.ShapeDtypeStruct((NC, N_P, HID), jnp.float32),
    mesh=_SC_MESH,
    scratch_types=[
        pltpu.VMEM((NIDX, 2, K), jnp.int32),
        pltpu.VMEM((NBUF, K, HID), jnp.float32),
        pltpu.VMEM_SHARED((N_P, HID), jnp.float32),
    ] + [pltpu.SemaphoreType.DMA] * (2 * NBUF + NIDX),
)


def _row_mask():
    return (lax.broadcasted_iota(jnp.int32, (N_P, 1), 0) < N_NODES
            ).astype(jnp.float32)


def _gcn_bn_relu(p_ref, g_ref, dinv_ref, b_ref, bng_ref, bnb_ref):
    mask = _row_mask()
    t = ((p_ref[0] + p_ref[1] - g_ref[...]) * dinv_ref[...] + b_ref[...]) * mask
    inv_n = 1.0 / N_NODES
    mu = jnp.sum(t, axis=0, keepdims=True) * inv_n
    d = (t - mu) * mask
    var = jnp.sum(d * d, axis=0, keepdims=True) * inv_n
    return jnp.maximum((t - mu) * lax.rsqrt(var + 1e-5) * bng_ref[...] + bnb_ref[...], 0.0)


def _tc0_body(x_ref, w_ref, dinv_ref, g_ref):
    g_ref[...] = jnp.dot(x_ref[...], w_ref[...],
                         preferred_element_type=jnp.float32) * dinv_ref[...]


def _tc_mid_body(p_ref, g_ref, dinv_ref, b_ref, bng_ref, bnb_ref, w_ref, out_ref):
    u = _gcn_bn_relu(p_ref, g_ref, dinv_ref, b_ref, bng_ref, bnb_ref)
    out_ref[...] = jnp.dot(u, w_ref[...],
                           preferred_element_type=jnp.float32) * dinv_ref[...]


def _tc_fin_body(p_ref, g_ref, dinv_ref, b_ref, bng_ref, bnb_ref, batch_ref, pf_ref,
                 pw1, pb1, pw2, pb2,
                 cw1a, cw1b, cb1, cw2, cb2,
                 aw1a0, aw1b0, ab10, aw20, ab20,
                 aw1a1, aw1b1, ab11, aw21, ab21,
                 logits_ref, a0_ref, a1_ref):
    u = _gcn_bn_relu(p_ref, g_ref, dinv_ref, b_ref, bng_ref, bnb_ref)

    onehot = (batch_ref[...] == lax.broadcasted_iota(jnp.int32, (1, NUM_GRAPHS), 1)
              ).astype(jnp.float32)
    dn = (((0,), (0,)), ((), ()))
    hp = jax.lax.Precision.HIGHEST
    sums = lax.dot_general(onehot, u, dn, precision=hp,
                           preferred_element_type=jnp.float32)
    counts = lax.dot_general(onehot, jnp.ones((N_P, 1), jnp.float32), dn, precision=hp,
                             preferred_element_type=jnp.float32)
    ligand = sums / jnp.maximum(counts, 1.0)

    pk = jnp.maximum(jnp.dot(pf_ref[...], pw1[...],
                             preferred_element_type=jnp.float32) + pb1[...], 0.0)
    pk2 = jnp.dot(pk, pw2[...], preferred_element_type=jnp.float32) + pb2[...]

    def head(w1a, w1b, b1, w2, b2):
        h = jnp.maximum(
            jnp.dot(ligand, w1a[...], preferred_element_type=jnp.float32)
            + jnp.dot(pk2, w1b[...], preferred_element_type=jnp.float32)
            + b1[...], 0.0)
        return jnp.dot(h, w2[...], preferred_element_type=jnp.float32) + b2[...]

    logits_ref[...] = head(cw1a, cw1b, cb1, cw2, cb2)
    a0_ref[...] = head(aw1a0, aw1b0, ab10, aw20, ab20)
    a1_ref[...] = head(aw1a1, aw1b1, ab11, aw21, ab21)


_tc0 = pl.pallas_call(
    _tc0_body, out_shape=jax.ShapeDtypeStruct((N_P, HID), jnp.float32))

_tc_mid = pl.pallas_call(
    _tc_mid_body, out_shape=jax.ShapeDtypeStruct((N_P, HID), jnp.float32))

_tc_fin = pl.pallas_call(
    _tc_fin_body,
    out_shape=[jax.ShapeDtypeStruct((NUM_GRAPHS, 1), jnp.float32)] * 3)


def kernel(x, edge_index, batch, pocket_features, params):
    src = edge_index[0]
    dst = edge_index[1]

    ei3 = jnp.stack([src.reshape(NW, NCHUNK, K), dst.reshape(NW, NCHUNK, K)],
                    axis=2)
    deg_parts = _deg_kernel(ei3, jnp.zeros((N_P,), jnp.float32),
                            jnp.ones((K,), jnp.float32))
    deg = deg_parts[0] + deg_parts[1] + 1.0
    dinv = lax.rsqrt(deg)[:, None]

    x_pad = jnp.pad(x, ((0, N_P - N_NODES), (0, 1)))
    w1_pad = jnp.pad(params["conv_w"][0], ((0, 1), (0, 0)))
    g = _tc0(x_pad, w1_pad, dinv)

    row = lambda v: v[None, :]
    for i in range(2):
        parts = _edge_kernel(g, ei3)
        g = _tc_mid(parts, g, dinv, row(params["conv_b"][i]),
                    row(params["bn_g"][i]), row(params["bn_b"][i]),
                    params["conv_w"][i + 1])

    parts = _edge_kernel(g, ei3)
    batch_pad = jnp.pad(batch, (0, N_P - N_NODES), constant_values=-1)
    logits, a0, a1 = _tc_fin(
        parts, g, dinv, row(params["conv_b"][2]),
        row(params["bn_g"][2]), row(params["bn_b"][2]),
        batch_pad[:, None], pocket_features[None, :],
        params["pocket_w1"], row(params["pocket_b1"]),
        params["pocket_w2"], row(params["pocket_b2"]),
        params["cls_w1"][:HID], params["cls_w1"][HID:], row(params["cls_b1"]),
        params["cls_w2"], row(params["cls_b2"]),
        params["aux_w1"][0][:HID], params["aux_w1"][0][HID:], row(params["aux_b1"][0]),
        params["aux_w2"][0], row(params["aux_b2"][0]),
        params["aux_w1"][1][:HID], params["aux_w1"][1][HID:], row(params["aux_b1"][1]),
        params["aux_w2"][1], row(params["aux_b2"][1]),
    )
    return logits[:, 0], a0[:, 0], a1[:, 0], jnp.float32(0.0)


# speedup vs baseline: 1.1395x; 1.0361x over previous
import functools

import jax
import jax.numpy as jnp
from jax import lax
from jax.experimental import pallas as pl
from jax.experimental.pallas import tpu as pltpu
from jax.experimental.pallas import tpu_sc as plsc

N_NODES = 10000
N_EDGES = 320000
NUM_GRAPHS = 256
HID = 128
NC, NS = 2, 16
NW = NC * NS
EPT = N_EDGES // NW
K = 40
NCHUNK = EPT // K
N_P = 10240
RPT = N_P // NS

_SC_MESH = plsc.VectorSubcoreMesh(core_axis_name="c", subcore_axis_name="s")


NBUF = 5
NIDX = 10
LOOK_G = 4
LOOK_I = 8
NTRIP = NCHUNK // NIDX


def _deg_body(ei3_hbm, zeros_hbm, ones_hbm, out_hbm, idxv, ones_v, acc, sem):
    c = lax.axis_index("c")
    s = lax.axis_index("s")
    wid = c * NS + s
    pltpu.sync_copy(ones_hbm, ones_v)
    pltpu.sync_copy(ei3_hbm.at[wid], idxv)
    pltpu.sync_copy(zeros_hbm.at[pl.ds(s * RPT, RPT)], acc.at[pl.ds(s * RPT, RPT)])
    plsc.subcore_barrier()

    def fire(i, carry):
        pltpu.async_copy(ones_v, acc.at[idxv.at[i, 1]], sem, add=True)
        return carry

    lax.fori_loop(0, NCHUNK, fire, 0)

    def drain(i, carry):
        pltpu.make_async_copy(ones_v, acc.at[idxv.at[0, 1]], sem).wait()
        return carry

    lax.fori_loop(0, NCHUNK, drain, 0)
    plsc.subcore_barrier()
    pltpu.sync_copy(acc.at[pl.ds(s * RPT, RPT)], out_hbm.at[c].at[pl.ds(s * RPT, RPT)])


_deg_kernel = pl.kernel(
    _deg_body,
    out_type=jax.ShapeDtypeStruct((NC, N_P), jnp.float32),
    mesh=_SC_MESH,
    scratch_types=[
        pltpu.VMEM((NCHUNK, 2, K), jnp.int32),
        pltpu.VMEM((K,), jnp.float32),
        pltpu.VMEM_SHARED((N_P,), jnp.float32),
        pltpu.SemaphoreType.DMA,
    ],
)


def _edge_body(g_hbm, ei3_hbm, out_hbm, idxv, rows, acc, *sems):
    gsem, ssem, isem = sems[:NBUF], sems[NBUF:2 * NBUF], sems[2 * NBUF:]
    c = lax.axis_index("c")
    s = lax.axis_index("s")
    wid = c * NS + s
    pltpu.sync_copy(g_hbm.at[pl.ds(s * RPT, RPT)], acc.at[pl.ds(s * RPT, RPT)])
    plsc.subcore_barrier()

    def idx_load(i, q):
        pltpu.async_copy(ei3_hbm.at[wid].at[i], idxv.at[q], isem[q])

    def idx_wait(q):
        pltpu.make_async_copy(ei3_hbm.at[wid].at[0], idxv.at[q], isem[q]).wait()

    def gather(i, q, b):
        pltpu.async_copy(g_hbm.at[idxv.at[q, 0]], rows.at[b], gsem[b])

    def gather_wait(b):
        pltpu.make_async_copy(g_hbm.at[idxv.at[0, 0]], rows.at[b], gsem[b]).wait()

    def scatter(i, q, b):
        pltpu.async_copy(rows.at[b], acc.at[idxv.at[q, 1]], ssem[b], add=True)

    def scatter_wait(b):
        pltpu.make_async_copy(rows.at[b], acc.at[idxv.at[0, 1]], ssem[b]).wait()

    for i in range(LOOK_I):
        idx_load(i, i)
    for i in range(LOOK_G):
        idx_wait(i)
        gather(i, i, i)

    def trip(t, carry):
        for u in range(NIDX):
            i = t * NIDX + u
            @pl.when(i + LOOK_I < NCHUNK)
            def _():
                idx_load(i + LOOK_I, (u + LOOK_I) % NIDX)

            bg = (u + LOOK_G) % NBUF
            @pl.when(jnp.logical_and(i + LOOK_G < NCHUNK, i + LOOK_G >= NBUF))
            def _():
                scatter_wait(bg)

            @pl.when(i + LOOK_G < NCHUNK)
            def _():
                idx_wait((u + LOOK_G) % NIDX)
                gather(i + LOOK_G, (u + LOOK_G) % NIDX, bg)

            gather_wait(u % NBUF)
            scatter(i, u % NIDX, u % NBUF)
        return carry

    lax.fori_loop(0, NTRIP, trip, 0)
    for b in range(NBUF):
        scatter_wait(b)
    plsc.subcore_barrier()
    pltpu.sync_copy(acc.at[pl.ds(s * RPT, RPT)], out_hbm.at[c].at[pl.ds(s * RPT, RPT)])


_edge_kernel = pl.kernel(
    _edge_body,
    out_type=jax.ShapeDtypeStruct((NC, N_P, HID), jnp.float32),
    mesh=_SC_MESH,
    scratch_types=[
        pltpu.VMEM((NIDX, 2, K), jnp.int32),
        pltpu.VMEM((NBUF, K, HID), jnp.float32),
        pltpu.VMEM_SHARED((N_P, HID), jnp.float32),
    ] + [pltpu.SemaphoreType.DMA] * (2 * NBUF + NIDX),
)


def _row_mask():
    return (lax.broadcasted_iota(jnp.int32, (N_P, 1), 0) < N_NODES
            ).astype(jnp.float32)


def _gcn_bn_relu(p_ref, g_ref, dinv_ref, b_ref, bng_ref, bnb_ref):
    mask = _row_mask()
    t = ((p_ref[0] + p_ref[1] - g_ref[...]) * dinv_ref[...] + b_ref[...]) * mask
    inv_n = 1.0 / N_NODES
    mu = jnp.sum(t, axis=0, keepdims=True) * inv_n
    d = (t - mu) * mask
    var = jnp.sum(d * d, axis=0, keepdims=True) * inv_n
    return jnp.maximum((t - mu) * lax.rsqrt(var + 1e-5) * bng_ref[...] + bnb_ref[...], 0.0)


def _tc0_body(x_ref, w_ref, dp_ref, g_ref, dinv_ref):
    dinv = lax.rsqrt(dp_ref[0] + dp_ref[1] + 1.0)
    dinv_ref[...] = dinv
    h = jnp.dot(x_ref[...], w_ref[...],
                preferred_element_type=jnp.float32)
    g_ref[pl.ds(0, N_NODES), :] = h * dinv[:N_NODES]
    g_ref[pl.ds(N_NODES, N_P - N_NODES), :] = jnp.zeros(
        (N_P - N_NODES, HID), jnp.float32)


def _tc_mid_body(p_ref, g_ref, dinv_ref, b_ref, bng_ref, bnb_ref, w_ref, out_ref):
    u = _gcn_bn_relu(p_ref, g_ref, dinv_ref, b_ref, bng_ref, bnb_ref)
    out_ref[...] = jnp.dot(u, w_ref[...],
                           preferred_element_type=jnp.float32) * dinv_ref[...]


def _tc_fin_body(p_ref, g_ref, dinv_ref, b_ref, bng_ref, bnb_ref, batch_ref, pf_ref,
                 pw1, pb1, pw2, pb2,
                 cw1a, cw1b, cb1, cw2, cb2,
                 aw1a0, aw1b0, ab10, aw20, ab20,
                 aw1a1, aw1b1, ab11, aw21, ab21,
                 logits_ref, a0_ref, a1_ref):
    u = _gcn_bn_relu(p_ref, g_ref, dinv_ref, b_ref, bng_ref, bnb_ref)

    onehot = (batch_ref[...] == lax.broadcasted_iota(jnp.int32, (1, NUM_GRAPHS), 1)
              ).astype(jnp.float32)
    dn = (((0,), (0,)), ((), ()))
    hp = jax.lax.Precision.HIGHEST
    sums = lax.dot_general(onehot, u[:N_NODES], dn, precision=hp,
                           preferred_element_type=jnp.float32)
    counts = lax.dot_general(onehot, jnp.ones((N_NODES, 1), jnp.float32), dn, precision=hp,
                             preferred_element_type=jnp.float32)
    ligand = sums / jnp.maximum(counts, 1.0)

    pk = jnp.maximum(jnp.dot(pf_ref[...], pw1[...],
                             preferred_element_type=jnp.float32) + pb1[...], 0.0)
    pk2 = jnp.dot(pk, pw2[...], preferred_element_type=jnp.float32) + pb2[...]

    def head(w1a, w1b, b1, w2, b2):
        h = jnp.maximum(
            jnp.dot(ligand, w1a[...], preferred_element_type=jnp.float32)
            + jnp.dot(pk2, w1b[...], preferred_element_type=jnp.float32)
            + b1[...], 0.0)
        return jnp.dot(h, w2[...], preferred_element_type=jnp.float32) + b2[...]

    logits_ref[...] = head(cw1a, cw1b, cb1, cw2, cb2)
    a0_ref[...] = head(aw1a0, aw1b0, ab10, aw20, ab20)
    a1_ref[...] = head(aw1a1, aw1b1, ab11, aw21, ab21)


_tc0 = pl.pallas_call(
    _tc0_body, out_shape=[jax.ShapeDtypeStruct((N_P, HID), jnp.float32),
                          jax.ShapeDtypeStruct((N_P, 1), jnp.float32)])

_tc_mid = pl.pallas_call(
    _tc_mid_body, out_shape=jax.ShapeDtypeStruct((N_P, HID), jnp.float32))

_tc_fin = pl.pallas_call(
    _tc_fin_body,
    out_shape=[jax.ShapeDtypeStruct((NUM_GRAPHS, 1), jnp.float32)] * 3)


def kernel(x, edge_index, batch, pocket_features, params):
    ei3 = edge_index.reshape(2, NW, NCHUNK, K).transpose(1, 2, 0, 3)
    deg_parts = _deg_kernel(ei3, jnp.zeros((N_P,), jnp.float32),
                            jnp.ones((K,), jnp.float32))
    g, dinv = _tc0(x, params["conv_w"][0], deg_parts[:, :, None])

    row = lambda v: v[None, :]
    for i in range(2):
        parts = _edge_kernel(g, ei3)
        g = _tc_mid(parts, g, dinv, row(params["conv_b"][i]),
                    row(params["bn_g"][i]), row(params["bn_b"][i]),
                    params["conv_w"][i + 1])

    parts = _edge_kernel(g, ei3)
    logits, a0, a1 = _tc_fin(
        parts, g, dinv, row(params["conv_b"][2]),
        row(params["bn_g"][2]), row(params["bn_b"][2]),
        batch[:, None], pocket_features[None, :],
        params["pocket_w1"], row(params["pocket_b1"]),
        params["pocket_w2"], row(params["pocket_b2"]),
        params["cls_w1"][:HID], params["cls_w1"][HID:], row(params["cls_b1"]),
        params["cls_w2"], row(params["cls_b2"]),
        params["aux_w1"][0][:HID], params["aux_w1"][0][HID:], row(params["aux_b1"][0]),
        params["aux_w2"][0], row(params["aux_b2"][0]),
        params["aux_w1"][1][:HID], params["aux_w1"][1][HID:], row(params["aux_b1"][1]),
        params["aux_w2"][1], row(params["aux_b2"][1]),
    )
    return logits[:, 0], a0[:, 0], a1[:, 0], jnp.float32(0.0)
